# Initial kernel scaffold; baseline (speedup 1.0000x reference)
import jax
import jax.numpy as jnp
from jax.experimental import pallas as pl


def kernel(x, W1l, b1l, W1r, W2l, b2l, W2r, Wa, ba, Wb, bb, edge_index, pos_edge_index, neg_edge_index):
    raise NotImplementedError("write your pallas kernel here")


# speedup vs baseline: 2.6511x; 2.6511x over previous
import functools

import jax
import jax.numpy as jnp
from jax import lax
from jax.experimental import pallas as pl
from jax.experimental.pallas import tpu as pltpu
from jax.experimental.pallas import tpu_sc as plsc

_NC = 2
_NS = 16
_NW = _NC * _NS
_LANES = 16

def _sc_mesh():
    return plsc.VectorSubcoreMesh(core_axis_name="c", subcore_axis_name="s",
                                  num_cores=_NC, num_subcores=_NS)


def _nwriters(n: int) -> int:
    nw = _NS
    while n % nw or (n // nw) % 8:
        nw -= 1
    return nw


def _pick_chunk(per_worker: int) -> int:
    for ec in range(128, 0, -8):
        if per_worker % ec == 0:
            return ec
    raise ValueError(per_worker)


@functools.lru_cache(maxsize=None)
def _build_segsum(n: int, d: int, e: int, interpret: bool = False):
    assert e % _NW == 0
    ew = e // _NW
    ec = _pick_chunk(ew)
    nchunks = ew // ec
    nwriters = _nwriters(n)
    rows_per_writer = n // nwriters

    def body(y_hbm, src_hbm, dst_hbm, zrows_hbm, zdeg_hbm, part_hbm, degp_hbm,
             sidx, didx, rows, ones, agg_sh, deg_sh, sem):
        c = lax.axis_index("c")
        s = lax.axis_index("s")
        w = s * _NC + c
        for j in range(ec // _LANES):
            ones[pl.ds(j * _LANES, _LANES)] = jnp.full((_LANES,), 1.0, jnp.float32)
        @pl.when(s < nwriters)
        def _():
            off = pl.multiple_of(s * rows_per_writer, 8)
            pltpu.sync_copy(zrows_hbm, agg_sh.at[pl.ds(off, rows_per_writer)])

        @pl.when(s == 0)
        def _():
            pltpu.sync_copy(zdeg_hbm, deg_sh)

        plsc.subcore_barrier()

        def chunk(i, carry):
            base = pl.multiple_of(w * ew + i * ec, 8)
            pltpu.sync_copy(src_hbm.at[pl.ds(base, ec)], sidx)
            pltpu.async_copy(y_hbm.at[sidx], rows, sem).wait()
            pltpu.sync_copy(dst_hbm.at[pl.ds(base, ec)], didx)
            pltpu.sync_copy(rows, agg_sh.at[didx], add=True)
            pltpu.sync_copy(ones, deg_sh.at[didx], add=True)
            return carry

        lax.fori_loop(0, nchunks, chunk, 0)
        plsc.subcore_barrier()

        @pl.when(s < nwriters)
        def _():
            off = pl.multiple_of(s * rows_per_writer, 8)
            pltpu.sync_copy(agg_sh.at[pl.ds(off, rows_per_writer)],
                            part_hbm.at[c, pl.ds(off, rows_per_writer)])

        @pl.when(s == 0)
        def _():
            pltpu.sync_copy(deg_sh, degp_hbm.at[c])

    return pl.kernel(
        body,
        out_type=(jax.ShapeDtypeStruct((_NC, n, d), jnp.float32),
                  jax.ShapeDtypeStruct((_NC, n), jnp.float32)),
        mesh=_sc_mesh(),
        scratch_types=[
            pltpu.VMEM((ec,), jnp.int32),
            pltpu.VMEM((ec,), jnp.int32),
            pltpu.VMEM((ec, d), jnp.float32),
            pltpu.VMEM((ec,), jnp.float32),
            pltpu.VMEM_SHARED((n, d), jnp.float32),
            pltpu.VMEM_SHARED((n,), jnp.float32),
            pltpu.SemaphoreType.DMA,
        ],
        compiler_params=pltpu.CompilerParams(needs_layout_passes=False),
        interpret=interpret,
    )


@functools.lru_cache(maxsize=None)
def _build_decode(n: int, d: int, ep: int, interpret: bool = False):
    assert ep % _NW == 0
    ew = ep // _NW
    ec = _pick_chunk(ew)
    nchunks = ew // ec

    def body(u_hbm, v_hbm, src_hbm, dst_hbm, wbb_hbm, out_hbm,
             sidx, didx, urows, vrows, wbv, outv, accbuf, sem):
        c = lax.axis_index("c")
        s = lax.axis_index("s")
        w = s * _NC + c
        pltpu.sync_copy(wbb_hbm, wbv)

        def chunk(i, carry):
            base = pl.multiple_of(w * ew + i * ec, 8)
            pltpu.sync_copy(src_hbm.at[pl.ds(base, ec)], sidx)
            pltpu.async_copy(u_hbm.at[sidx], urows, sem).wait()
            pltpu.sync_copy(dst_hbm.at[pl.ds(base, ec)], didx)
            pltpu.async_copy(v_hbm.at[didx], vrows, sem).wait()
            bb = wbv[pl.ds(d, _LANES)][0]
            lane = lax.iota(jnp.int32, _LANES)

            def group(g, carry2):
                for l in range(_LANES):
                    e2 = g * _LANES + l
                    acc = jnp.zeros((_LANES,), jnp.float32)
                    for j in range(d // _LANES):
                        t = jnp.maximum(
                            urows[e2, pl.ds(j * _LANES, _LANES)]
                            + vrows[e2, pl.ds(j * _LANES, _LANES)], 0.0)
                        acc = acc + t * wbv[pl.ds(j * _LANES, _LANES)]
                    accbuf[e2, :] = acc
                rowidx = g * _LANES + lane
                res = jnp.zeros((_LANES,), jnp.float32)
                for l in range(_LANES):
                    col = jnp.full((_LANES,), l, jnp.int32)
                    res = res + plsc.load_gather(accbuf, [rowidx, col])
                outv[pl.ds(g * _LANES, _LANES)] = res + bb
                return carry2

            lax.fori_loop(0, ec // _LANES, group, 0)
            pltpu.sync_copy(outv, out_hbm.at[pl.ds(base, ec)])
            return carry

        lax.fori_loop(0, nchunks, chunk, 0)

    return pl.kernel(
        body,
        out_type=jax.ShapeDtypeStruct((ep,), jnp.float32),
        mesh=_sc_mesh(),
        scratch_types=[
            pltpu.VMEM((ec,), jnp.int32),
            pltpu.VMEM((ec,), jnp.int32),
            pltpu.VMEM((ec, d), jnp.float32),
            pltpu.VMEM((ec, d), jnp.float32),
            pltpu.VMEM((d + _LANES,), jnp.float32),
            pltpu.VMEM((ec,), jnp.float32),
            pltpu.VMEM((ec, _LANES), jnp.float32),
            pltpu.SemaphoreType.DMA,
        ],
        compiler_params=pltpu.CompilerParams(needs_layout_passes=False),
        interpret=interpret,
    )


_RB = 1000


def _combine1_body(p_ref, degb_ref, x_ref, w1lt_ref, b1l_ref, w1rt_ref, o_ref):
    dsum = jnp.sum(degb_ref[...], axis=1)
    dinv = 1.0 / jnp.maximum(dsum, 1.0)
    agg = (p_ref[0] + p_ref[1]) * dinv[:, None]
    z = (jnp.dot(agg, w1lt_ref[...], preferred_element_type=jnp.float32)
         + b1l_ref[...]
         + jnp.dot(x_ref[...], w1rt_ref[...], preferred_element_type=jnp.float32))
    o_ref[...] = jnp.maximum(z, 0.0)


def _combine2_body(q_ref, degb_ref, z1_ref, w2lt_ref, b2l_ref, w2rt_ref,
                   walt_ref, ba_ref, wart_ref, u_ref, v_ref):
    dsum = jnp.sum(degb_ref[...], axis=1)
    dinv = 1.0 / jnp.maximum(dsum, 1.0)
    agg = (q_ref[0] + q_ref[1]) * dinv[:, None]
    z = (jnp.dot(agg, w2lt_ref[...], preferred_element_type=jnp.float32)
         + b2l_ref[...]
         + jnp.dot(z1_ref[...], w2rt_ref[...], preferred_element_type=jnp.float32))
    u_ref[...] = jnp.dot(z, walt_ref[...], preferred_element_type=jnp.float32) + ba_ref[...]
    v_ref[...] = jnp.dot(z, wart_ref[...], preferred_element_type=jnp.float32)


@functools.lru_cache(maxsize=None)
def _build_combine1(n: int, d: int, interpret: bool = False):
    nb = n // _RB
    return pl.pallas_call(
        _combine1_body,
        grid=(nb,),
        in_specs=[
            pl.BlockSpec((_NC, _RB, d), lambda i: (0, i, 0)),
            pl.BlockSpec((_RB, _NC), lambda i: (i, 0)),
            pl.BlockSpec((_RB, d), lambda i: (i, 0)),
            pl.BlockSpec((d, d), lambda i: (0, 0)),
            pl.BlockSpec((1, d), lambda i: (0, 0)),
            pl.BlockSpec((d, d), lambda i: (0, 0)),
        ],
        out_specs=pl.BlockSpec((_RB, d), lambda i: (i, 0)),
        out_shape=jax.ShapeDtypeStruct((n, d), jnp.float32),
        interpret=interpret,
    )


@functools.lru_cache(maxsize=None)
def _build_combine2(n: int, d: int, interpret: bool = False):
    nb = n // _RB
    return pl.pallas_call(
        _combine2_body,
        grid=(nb,),
        in_specs=[
            pl.BlockSpec((_NC, _RB, d), lambda i: (0, i, 0)),
            pl.BlockSpec((_RB, _NC), lambda i: (i, 0)),
            pl.BlockSpec((_RB, d), lambda i: (i, 0)),
            pl.BlockSpec((d, d), lambda i: (0, 0)),
            pl.BlockSpec((1, d), lambda i: (0, 0)),
            pl.BlockSpec((d, d), lambda i: (0, 0)),
            pl.BlockSpec((d, d), lambda i: (0, 0)),
            pl.BlockSpec((1, d), lambda i: (0, 0)),
            pl.BlockSpec((d, d), lambda i: (0, 0)),
        ],
        out_specs=[pl.BlockSpec((_RB, d), lambda i: (i, 0)),
                   pl.BlockSpec((_RB, d), lambda i: (i, 0))],
        out_shape=[jax.ShapeDtypeStruct((n, d), jnp.float32),
                   jax.ShapeDtypeStruct((n, d), jnp.float32)],
        interpret=interpret,
    )


def _run(x, W1l, b1l, W1r, W2l, b2l, W2r, Wa, ba, Wb, bb,
         edge_index, pos_edge_index, neg_edge_index, interpret=False):
    n, d = x.shape
    e = edge_index.shape[1]
    pe = pos_edge_index.shape[1]
    segsum = _build_segsum(n, d, e, interpret)
    combine1 = _build_combine1(n, d, interpret)
    combine2 = _build_combine2(n, d, interpret)

    src = edge_index[0]
    dst = edge_index[1]
    zrows = jnp.zeros((n // _nwriters(n), d), jnp.float32)
    zdeg = jnp.zeros((n,), jnp.float32)

    part1, degp = segsum(x, src, dst, zrows, zdeg)
    degb = degp.T
    z1 = combine1(part1, degb, x, W1l.T, b1l[None, :], W1r.T)
    part2, _ = segsum(z1, src, dst, zrows, zdeg)
    u, v = combine2(part2, degb, z1, W2l.T, b2l[None, :], W2r.T,
                    Wa[:, :d].T, ba[None, :], Wa[:, d:].T)

    ntot = 2 * pe
    ep = ((ntot + 8 * _NW - 1) // (8 * _NW)) * (8 * _NW)
    pad = jnp.zeros((ep - ntot,), src.dtype)
    srcp = jnp.concatenate([pos_edge_index[0], neg_edge_index[0], pad])
    dstp = jnp.concatenate([pos_edge_index[1], neg_edge_index[1], pad])
    wbb = jnp.concatenate([Wb[0], jnp.full((_LANES,), bb[0], jnp.float32)])
    decode = _build_decode(n, d, ep, interpret)
    preds = decode(u, v, srcp, dstp, wbb)
    return preds[:pe], preds[pe:2 * pe]


def kernel(x, W1l, b1l, W1r, W2l, b2l, W2r, Wa, ba, Wb, bb,
           edge_index, pos_edge_index, neg_edge_index):
    return _run(x, W1l, b1l, W1r, W2l, b2l, W2r, Wa, ba, Wb, bb,
                edge_index, pos_edge_index, neg_edge_index)
